# baseline (speedup 1.0000x reference)
import dataclasses
import functools

import jax
import jax.numpy as jnp
from jax import lax
from jax.experimental import pallas as pl
from jax.experimental.pallas import tpu as pltpu
from jax.experimental.pallas import tpu_sc as plsc

N = 10000
E = 320000
H = 512
G = 16
OUT = 128
NB = 10
R = N // NB

NC = 2
NS = 16
NW = NC * NS
NSP = 32
ES = E // NSP
RNG = 160
NR = 2
NRANGES = NW * NR
AGGR = NRANGES * RNG
HGL = 80
KMAX = 80
CH = 16
HPAD = 11000
ZROW = N
BATCH = 64
SELCAP = ES + 256
RESCAP = 11264
FCH = 1024

_vector_mesh = plsc.VectorSubcoreMesh(core_axis_name="c", subcore_axis_name="s")

_sc_params = pltpu.CompilerParams()
if "needs_layout_passes" in pltpu.CompilerParams.__dataclass_fields__:
    _sc_params = dataclasses.replace(_sc_params, needs_layout_passes=False)


def _bucket_body(src_hbm, dst_hbm, ell_hbm, rs_hbm, rd_hbm, cnts_hbm,
                 kmax_hbm, degp_hbm,
                 sbuf, dbuf, sel_s, sel_d, res_s, res_d, ell_st, cnt_l,
                 deg_l, cv32, kv):
    cid = lax.axis_index("c")
    sid = lax.axis_index("s")
    w = cid * NS + sid

    zero16f = jnp.zeros((16,), jnp.float32)
    ones16 = jnp.ones((16,), jnp.float32)
    zero16i = jnp.zeros((16,), jnp.int32)
    zrow16 = jnp.full((16,), ZROW, jnp.int32)
    sent16 = jnp.full((16,), RNG, jnp.int32)
    iota16 = lax.iota(jnp.int32, 16)
    lane0 = iota16 == 0

    for r in range(NR):
        rid = r * NW + w
        lo = rid * RNG

        @pl.loop(0, RNG // 16)
        def _(i):
            deg_l[pl.ds(i * 16, 16)] = zero16f

        @pl.loop(0, (RNG + 16) // 16)
        def _(i):
            cnt_l[pl.ds(i * 16, 16)] = zero16i

        @pl.loop(0, KMAX)
        def _(k):
            @pl.loop(0, 2)
            def _(p):
                @pl.loop(0, HGL // 16)
                def _(q):
                    ell_st[k, p, pl.ds(q * 16, 16)] = zrow16

        def span_step(s, cvs, lo=lo, rid=rid):
            cv0, cv1 = cvs
            pltpu.sync_copy(src_hbm.at[pl.ds(s * ES, ES)], sbuf)
            pltpu.sync_copy(dst_hbm.at[pl.ds(s * ES, ES)], dbuf)

            def step(i, cur):
                sv = sbuf[pl.ds(i * 16, 16)]
                dv = dbuf[pl.ds(i * 16, 16)] - lo
                m = (dv >= 0) & (dv < RNG)
                plsc.store_compressed(sel_s.at[pl.ds(cur, 16)], sv, mask=m)
                plsc.store_compressed(sel_d.at[pl.ds(cur, 16)], dv, mask=m)
                plsc.addupdate_scatter(deg_l, [dv], ones16, mask=m)
                return cur + jnp.max(plsc.all_reduce_population_count(m))

            cur = lax.fori_loop(0, ES // 16, step, jnp.int32(0))

            def walk(e, rcur):
                ld = sel_d[pl.ds(e, 16)][0]
                sv = sel_s[pl.ds(e, 16)][0]
                c = cnt_l[pl.ds(ld, 16)][0]
                ldspl = jnp.full((16,), ld, jnp.int32)
                svspl = jnp.full((16,), sv, jnp.int32)

                @pl.when(c < KMAX)
                def _():
                    ph = ld // HGL
                    col = ld - ph * HGL
                    plsc.store_scatter(
                        ell_st,
                        [jnp.full((16,), c, jnp.int32),
                         jnp.full((16,), ph, jnp.int32),
                         jnp.full((16,), col, jnp.int32)],
                        svspl, mask=lane0)
                    plsc.store_scatter(
                        cnt_l, [ldspl], jnp.full((16,), c + 1, jnp.int32),
                        mask=lane0)

                @pl.when(c >= KMAX)
                def _():
                    plsc.store_compressed(res_s.at[pl.ds(rcur, 16)], svspl,
                                          mask=lane0)
                    plsc.store_compressed(res_d.at[pl.ds(rcur, 16)], ldspl,
                                          mask=lane0)

                return rcur + jnp.where(c >= KMAX, 1, 0).astype(jnp.int32)

            rcur = lax.fori_loop(0, cur, walk, jnp.int32(0))

            for k in range(BATCH // 16):
                res_s[pl.ds(rcur + k * 16, 16)] = zrow16
                res_d[pl.ds(rcur + k * 16, 16)] = zero16i
            npad = ((rcur + BATCH - 1) // BATCH) * BATCH

            def flush(j, carry):
                pltpu.sync_copy(res_s.at[pl.ds(j * FCH, FCH)],
                                rs_hbm.at[rid, s, pl.ds(j * FCH, FCH)])
                pltpu.sync_copy(res_d.at[pl.ds(j * FCH, FCH)],
                                rd_hbm.at[rid, s, pl.ds(j * FCH, FCH)])
                return carry

            lax.fori_loop(0, (npad + FCH - 1) // FCH, flush, jnp.int32(0))
            sm = iota16 == (s % 16)
            cv0 = jnp.where(sm & (s < 16), npad, cv0)
            cv1 = jnp.where(sm & (s >= 16), npad, cv1)
            return (cv0, cv1)

        cv0, cv1 = lax.fori_loop(0, NSP, span_step, (zero16i, zero16i))
        cv32[pl.ds(0, 16)] = cv0
        cv32[pl.ds(16, 16)] = cv1
        pltpu.sync_copy(cv32, cnts_hbm.at[rid])

        def mx(i, m):
            return jnp.maximum(m, cnt_l[pl.ds(i * 16, 16)])

        mv = lax.fori_loop(0, RNG // 16, mx, zero16i)
        kmax = jnp.maximum(jnp.max(mv), 1)
        kv[...] = jnp.where(lane0, kmax, 0)
        pltpu.sync_copy(kv, kmax_hbm.at[rid])
        pltpu.sync_copy(ell_st, ell_hbm.at[rid])
        pltpu.sync_copy(deg_l, degp_hbm.at[rid])


@functools.partial(
    pl.kernel,
    out_type=(
        jax.ShapeDtypeStruct((NRANGES, KMAX, 2, HGL), jnp.int32),
        jax.ShapeDtypeStruct((NRANGES, NSP, RESCAP), jnp.int32),
        jax.ShapeDtypeStruct((NRANGES, NSP, RESCAP), jnp.int32),
        jax.ShapeDtypeStruct((NRANGES, 32), jnp.int32),
        jax.ShapeDtypeStruct((NRANGES, 16), jnp.int32),
        jax.ShapeDtypeStruct((NRANGES, RNG), jnp.float32),
    ),
    mesh=_vector_mesh,
    scratch_types=[
        pltpu.VMEM((ES,), jnp.int32),
        pltpu.VMEM((ES,), jnp.int32),
        pltpu.VMEM((SELCAP,), jnp.int32),
        pltpu.VMEM((SELCAP,), jnp.int32),
        pltpu.VMEM((RESCAP,), jnp.int32),
        pltpu.VMEM((RESCAP,), jnp.int32),
        pltpu.VMEM((KMAX, 2, HGL), jnp.int32),
        pltpu.VMEM((RNG + 16,), jnp.int32),
        pltpu.VMEM((RNG,), jnp.float32),
        pltpu.VMEM((32,), jnp.int32),
        pltpu.VMEM((16,), jnp.int32),
    ],
    compiler_params=_sc_params,
)
def _bucket(*args):
    _bucket_body(*args)


NG = H // 128


def _agg_body(h_hbm, h4_hbm, ell_hbm, rs_hbm, rd_hbm, cnts_hbm, kmax_hbm,
              agg_hbm,
              aA0, aA1, aA2, aA3, aB0, aB1, aB2, aB3,
              rows, ebuf, i0, i1, i2, i3, j0, j1, j2, j3,
              sidx, didx, cv32, kv, semA, semB):
    cid = lax.axis_index("c")
    sid = lax.axis_index("s")
    w = cid * NS + sid
    iota16 = lax.iota(jnp.int32, 16)
    zero16f = jnp.zeros((16,), jnp.float32)
    accsA = [aA0, aA1, aA2, aA3]
    accsB = [aB0, aB1, aB2, aB3]
    ibs = [i0, i1, i2, i3]
    jbs = [j0, j1, j2, j3]

    for r in range(NR):
        rid = r * NW + w
        lo = rid * RNG

        pltpu.sync_copy(kmax_hbm.at[rid], kv)
        kmax = jnp.max(jnp.where(iota16 == 0, kv[...], 0))

        @pl.loop(0, HGL)
        def _(i):
            @pl.loop(0, 128 // 16)
            def _(q):
                for g in range(NG):
                    accsA[g][i, pl.ds(q * 16, 16)] = zero16f
                    accsB[g][i, pl.ds(q * 16, 16)] = zero16f

        def chunk_step(c, carry, rid=rid):
            pltpu.sync_copy(ell_hbm.at[rid, pl.ds(c * CH, CH)], ebuf)
            nk = jnp.minimum(CH, kmax - c * CH)

            def round_step(k, c2):
                for q in range(HGL // 16):
                    e0 = ebuf[k, 0, pl.ds(q * 16, 16)] * NG
                    e1 = ebuf[k, 1, pl.ds(q * 16, 16)] * NG
                    for g in range(NG):
                        ibs[g][pl.ds(q * 16, 16)] = e0 + g
                        jbs[g][pl.ds(q * 16, 16)] = e1 + g
                ds_ = []
                for g in range(NG):
                    ds_.append(pltpu.async_copy(h4_hbm.at[ibs[g]], accsA[g],
                                                semA, add=True))
                    ds_.append(pltpu.async_copy(h4_hbm.at[jbs[g]], accsB[g],
                                                semB, add=True))
                for d in ds_:
                    d.wait()
                return c2

            lax.fori_loop(0, nk, round_step, jnp.int32(0))
            return carry

        lax.fori_loop(0, (kmax + CH - 1) // CH, chunk_step, jnp.int32(0))

        pltpu.sync_copy(cnts_hbm.at[rid], cv32)

        def span_step(s, carry, rid=rid):
            nsel = cv32[pl.ds((s // 16) * 16, 16)]
            n = jnp.max(jnp.where(iota16 == s % 16, nsel, 0))

            def batch_step(j, c2):
                off = j * BATCH
                pltpu.sync_copy(rs_hbm.at[rid, s, pl.ds(off, BATCH)], sidx)
                pltpu.sync_copy(rd_hbm.at[rid, s, pl.ds(off, BATCH)],
                                didx.at[pl.ds(0, BATCH)])
                pltpu.async_copy(h_hbm.at[sidx], rows, semA).wait()

                def edge_step(e, c3):
                    ld = didx[pl.ds(e, 16)][0]
                    ldspl = jnp.full((16,), ld, jnp.int32)
                    mA = ldspl < HGL
                    for k in range(H // 16):
                        x = rows[e, pl.ds(k * 16, 16)]
                        g, q = k // 8, k % 8
                        col = iota16 + (q * 16)
                        plsc.addupdate_scatter(accsA[g], [ldspl, col], x,
                                               mask=mA)
                        plsc.addupdate_scatter(accsB[g], [ldspl - HGL, col],
                                               x, mask=~mA)
                    return c3

                lax.fori_loop(0, BATCH, edge_step, jnp.int32(0))
                return c2

            lax.fori_loop(0, n // BATCH, batch_step, jnp.int32(0))
            return carry

        lax.fori_loop(0, NSP, span_step, jnp.int32(0))
        for g in range(NG):
            pltpu.sync_copy(accsA[g], agg_hbm.at[g, pl.ds(lo, HGL)])
            pltpu.sync_copy(accsB[g], agg_hbm.at[g, pl.ds(lo + HGL, HGL)])


@functools.partial(
    pl.kernel,
    out_type=jax.ShapeDtypeStruct((NG, AGGR, 128), jnp.float32),
    mesh=_vector_mesh,
    scratch_types=(
        [pltpu.VMEM((HGL, 128), jnp.float32) for _ in range(2 * NG)] +
        [pltpu.VMEM((BATCH, H), jnp.float32),
         pltpu.VMEM((CH, 2, HGL), jnp.int32)] +
        [pltpu.VMEM((HGL,), jnp.int32) for _ in range(2 * NG)] +
        [pltpu.VMEM((BATCH,), jnp.int32),
         pltpu.VMEM((BATCH + 16,), jnp.int32),
         pltpu.VMEM((32,), jnp.int32),
         pltpu.VMEM((16,), jnp.int32),
         pltpu.SemaphoreType.DMA,
         pltpu.SemaphoreType.DMA]
    ),
    compiler_params=_sc_params,
)
def _agg(*args):
    _agg_body(*args)


def _mm_stats_body(h_ref, a0_ref, a1_ref, a2_ref, a3_ref, dg_ref,
                   wl_ref, wr_ref, bc_ref, u_ref, stats_ref):
    i = pl.program_id(0)
    h = h_ref[...]
    dg = dg_ref[0, 0, :]
    a = jnp.concatenate([a0_ref[0], a1_ref[0], a2_ref[0], a3_ref[0]],
                        axis=1) * dg[:, None]
    u = lax.dot_general(h, wl_ref[...], (((1,), (1,)), ((), ())),
                        preferred_element_type=jnp.float32)
    u += lax.dot_general(a, wr_ref[...], (((1,), (1,)), ((), ())),
                         preferred_element_type=jnp.float32)
    u += bc_ref[...]
    u_ref[...] = u
    s1 = jnp.sum(u, axis=0, keepdims=True)
    s2 = jnp.sum(u * u, axis=0, keepdims=True)
    new = jnp.concatenate([s1, s2, jnp.zeros((6, H), jnp.float32)], axis=0)

    @pl.when(i == 0)
    def _():
        stats_ref[...] = new

    @pl.when(i > 0)
    def _():
        stats_ref[...] += new


def _mm_stats(h, agg4, deginv3d, wl, wr, bc):
    return pl.pallas_call(
        _mm_stats_body,
        grid=(NB,),
        in_specs=[
            pl.BlockSpec((R, H), lambda i: (i, 0)),
            pl.BlockSpec((1, R, 128), lambda i: (0, i, 0)),
            pl.BlockSpec((1, R, 128), lambda i: (1, i, 0)),
            pl.BlockSpec((1, R, 128), lambda i: (2, i, 0)),
            pl.BlockSpec((1, R, 128), lambda i: (3, i, 0)),
            pl.BlockSpec((1, 1, R), lambda i: (i, 0, 0)),
            pl.BlockSpec((H, H), lambda i: (0, 0)),
            pl.BlockSpec((H, H), lambda i: (0, 0)),
            pl.BlockSpec((1, H), lambda i: (0, 0)),
        ],
        out_specs=[
            pl.BlockSpec((R, H), lambda i: (i, 0)),
            pl.BlockSpec((8, H), lambda i: (0, 0)),
        ],
        out_shape=[
            jax.ShapeDtypeStruct((N, H), jnp.float32),
            jax.ShapeDtypeStruct((8, H), jnp.float32),
        ],
    )(h, agg4, agg4, agg4, agg4, deginv3d, wl, wr, bc)


def _bn_body(u_ref, stats_ref, g_ref, b_ref, o_ref):
    i = pl.program_id(0)

    @pl.when(i < NB)
    def _():
        u = u_ref[...]
        mu = stats_ref[0:1, :] * (1.0 / N)
        var = stats_ref[1:2, :] * (1.0 / N) - mu * mu
        inv = lax.rsqrt(var + 1e-5)
        o_ref[...] = jnp.maximum((u - mu) * inv * g_ref[...] + b_ref[...],
                                 0.0)

    @pl.when(i == NB)
    def _():
        o_ref[...] = jnp.zeros((R, H), jnp.float32)


def _bn_relu(u, stats, g, b):
    return pl.pallas_call(
        _bn_body,
        grid=(NB + 1,),
        in_specs=[
            pl.BlockSpec((R, H), lambda i: (jnp.minimum(i, NB - 1), 0)),
            pl.BlockSpec((8, H), lambda i: (0, 0)),
            pl.BlockSpec((1, H), lambda i: (0, 0)),
            pl.BlockSpec((1, H), lambda i: (0, 0)),
        ],
        out_specs=pl.BlockSpec((R, H), lambda i: (i, 0)),
        out_shape=jax.ShapeDtypeStruct((HPAD, H), jnp.float32),
    )(u, stats, g, b)


def _head_body(h_ref, batch_ref, fcw_ref, fcb_ref, w1_ref, b1_ref,
               w2_ref, b2_ref, o_ref, pooled_acc, cnt_acc):
    i = pl.program_id(0)
    b = batch_ref[0, 0, :]
    onehot = (b[:, None] == lax.broadcasted_iota(jnp.int32, (1, G), 1)
              ).astype(jnp.float32)
    pooled = lax.dot_general(onehot, h_ref[...], (((0,), (0,)), ((), ())),
                             preferred_element_type=jnp.float32)
    cnt = jnp.sum(onehot, axis=0, keepdims=True)

    @pl.when(i == 0)
    def _():
        pooled_acc[...] = pooled
        cnt_acc[...] = cnt

    @pl.when(i > 0)
    def _():
        pooled_acc[...] += pooled
        cnt_acc[...] += cnt

    @pl.when(i == NB - 1)
    def _():
        p = pooled_acc[...] / jnp.maximum(cnt_acc[...], 1.0).reshape(G, 1)
        t = lax.dot_general(p, fcw_ref[...], (((1,), (1,)), ((), ())),
                            preferred_element_type=jnp.float32)
        t = jnp.maximum(t + fcb_ref[...], 0.0)
        t = lax.dot_general(t, w1_ref[...], (((1,), (1,)), ((), ())),
                            preferred_element_type=jnp.float32)
        t = jnp.maximum(t + b1_ref[...], 0.0)
        t = lax.dot_general(t, w2_ref[...], (((1,), (1,)), ((), ())),
                            preferred_element_type=jnp.float32)
        o_ref[...] = t + b2_ref[...]


def _pool_head(h, batch3d, fc_W, fc_b, w1, b1, w2, b2):
    return pl.pallas_call(
        _head_body,
        grid=(NB,),
        in_specs=[
            pl.BlockSpec((R, H), lambda i: (i, 0)),
            pl.BlockSpec((1, 1, R), lambda i: (i, 0, 0)),
            pl.BlockSpec((H, H), lambda i: (0, 0)),
            pl.BlockSpec((1, H), lambda i: (0, 0)),
            pl.BlockSpec((H, H), lambda i: (0, 0)),
            pl.BlockSpec((1, H), lambda i: (0, 0)),
            pl.BlockSpec((OUT, H), lambda i: (0, 0)),
            pl.BlockSpec((1, OUT), lambda i: (0, 0)),
        ],
        out_specs=pl.BlockSpec((G, OUT), lambda i: (0, 0)),
        out_shape=jax.ShapeDtypeStruct((G, OUT), jnp.float32),
        scratch_shapes=[
            pltpu.VMEM((G, H), jnp.float32),
            pltpu.VMEM((1, G), jnp.float32),
        ],
    )(h, batch3d, fc_W, fc_b, w1, b1, w2, b2)


def kernel(x, gam0, gam1, gam2, edge_index, batch,
           Wl0, Wr0, bc0, bng0, bnb0,
           Wl1, Wr1, bc1, bng1, bnb1,
           Wl2, Wr2, bc2, bng2, bnb2,
           Wl3, Wr3, bc3, bng3, bnb3,
           fc_W, fc_b, mlp_W1, mlp_b1, mlp_W2, mlp_b2):
    h = jnp.concatenate([x, gam0, gam1, gam2], axis=1)
    h = jnp.concatenate([h, jnp.zeros((HPAD - N, H), jnp.float32)], axis=0)
    src = edge_index[0].astype(jnp.int32)
    dst = edge_index[1].astype(jnp.int32)
    batch3d = batch.astype(jnp.int32).reshape(NB, 1, R)

    ell, rs, rd, cnts, kmaxs, degp = _bucket(src, dst)
    deg = degp.reshape(AGGR)[:N]
    deginv3d = (1.0 / jnp.clip(deg, 1.0)).reshape(NB, 1, R)

    convs = [(Wl0, Wr0, bc0, bng0, bnb0), (Wl1, Wr1, bc1, bng1, bnb1),
             (Wl2, Wr2, bc2, bng2, bnb2), (Wl3, Wr3, bc3, bng3, bnb3)]
    for (Wl, Wr, bc, g, b) in convs:
        agg4 = _agg(h, h.reshape(HPAD * NG, 128), ell, rs, rd, cnts, kmaxs)
        u, stats = _mm_stats(h, agg4, deginv3d, Wl, Wr, bc.reshape(1, H))
        h = _bn_relu(u, stats, g.reshape(1, H), b.reshape(1, H))

    return _pool_head(h, batch3d, fc_W, fc_b.reshape(1, H),
                      mlp_W1, mlp_b1.reshape(1, H),
                      mlp_W2, mlp_b2.reshape(1, OUT))


# speedup vs baseline: 1.0284x; 1.0284x over previous
import dataclasses
import functools

import jax
import jax.numpy as jnp
from jax import lax
from jax.experimental import pallas as pl
from jax.experimental.pallas import tpu as pltpu
from jax.experimental.pallas import tpu_sc as plsc

N = 10000
E = 320000
H = 512
G = 16
OUT = 128
NB = 10
R = N // NB

NC = 2
NS = 16
NW = NC * NS
NSP = 32
ES = E // NSP
RNG = 160
NR = 2
NRANGES = NW * NR
AGGR = NRANGES * RNG
HGL = 80
KMAX = 80
CH = 16
HPAD = 11000
ZROW = N
BATCH = 64
SELCAP = ES + 256
RESCAP = 11264
FCH = 1024

_vector_mesh = plsc.VectorSubcoreMesh(core_axis_name="c", subcore_axis_name="s")

_sc_params = pltpu.CompilerParams()
if "needs_layout_passes" in pltpu.CompilerParams.__dataclass_fields__:
    _sc_params = dataclasses.replace(_sc_params, needs_layout_passes=False)


def _bucket_body(src_hbm, dst_hbm, ell_hbm, rs_hbm, rd_hbm, cnts_hbm,
                 kmax_hbm, degp_hbm,
                 sbuf, dbuf, sel_s, sel_d, res_s, res_d, ell_st, cnt_l,
                 deg_l, cv32, kv):
    cid = lax.axis_index("c")
    sid = lax.axis_index("s")
    w = cid * NS + sid

    zero16f = jnp.zeros((16,), jnp.float32)
    ones16 = jnp.ones((16,), jnp.float32)
    zero16i = jnp.zeros((16,), jnp.int32)
    zrow16 = jnp.full((16,), ZROW, jnp.int32)
    sent16 = jnp.full((16,), RNG, jnp.int32)
    iota16 = lax.iota(jnp.int32, 16)
    lane0 = iota16 == 0

    for r in range(NR):
        rid = r * NW + w
        lo = rid * RNG

        @pl.loop(0, RNG // 16)
        def _(i):
            deg_l[pl.ds(i * 16, 16)] = zero16f

        @pl.loop(0, (RNG + 16) // 16)
        def _(i):
            cnt_l[pl.ds(i * 16, 16)] = zero16i

        @pl.loop(0, KMAX)
        def _(k):
            @pl.loop(0, 2)
            def _(p):
                @pl.loop(0, HGL // 16)
                def _(q):
                    ell_st[k, p, pl.ds(q * 16, 16)] = zrow16

        def span_step(s, cvs, lo=lo, rid=rid):
            cv0, cv1 = cvs
            pltpu.sync_copy(src_hbm.at[pl.ds(s * ES, ES)], sbuf)
            pltpu.sync_copy(dst_hbm.at[pl.ds(s * ES, ES)], dbuf)

            def step(i, cur):
                sv = sbuf[pl.ds(i * 16, 16)]
                dv = dbuf[pl.ds(i * 16, 16)] - lo
                m = (dv >= 0) & (dv < RNG)
                plsc.store_compressed(sel_s.at[pl.ds(cur, 16)], sv, mask=m)
                plsc.store_compressed(sel_d.at[pl.ds(cur, 16)], dv, mask=m)
                plsc.addupdate_scatter(deg_l, [dv], ones16, mask=m)
                return cur + jnp.max(plsc.all_reduce_population_count(m))

            cur = lax.fori_loop(0, ES // 16, step, jnp.int32(0))

            def walk(e, rcur):
                ld = sel_d[pl.ds(e, 16)][0]
                sv = sel_s[pl.ds(e, 16)][0]
                c = cnt_l[pl.ds(ld, 16)][0]
                ldspl = jnp.full((16,), ld, jnp.int32)
                svspl = jnp.full((16,), sv, jnp.int32)

                @pl.when(c < KMAX)
                def _():
                    ph = ld // HGL
                    col = ld - ph * HGL
                    plsc.store_scatter(
                        ell_st,
                        [jnp.full((16,), c, jnp.int32),
                         jnp.full((16,), ph, jnp.int32),
                         jnp.full((16,), col, jnp.int32)],
                        svspl, mask=lane0)
                    plsc.store_scatter(
                        cnt_l, [ldspl], jnp.full((16,), c + 1, jnp.int32),
                        mask=lane0)

                @pl.when(c >= KMAX)
                def _():
                    plsc.store_compressed(res_s.at[pl.ds(rcur, 16)], svspl,
                                          mask=lane0)
                    plsc.store_compressed(res_d.at[pl.ds(rcur, 16)], ldspl,
                                          mask=lane0)

                return rcur + jnp.where(c >= KMAX, 1, 0).astype(jnp.int32)

            rcur = lax.fori_loop(0, cur, walk, jnp.int32(0))

            for k in range(BATCH // 16):
                res_s[pl.ds(rcur + k * 16, 16)] = zrow16
                res_d[pl.ds(rcur + k * 16, 16)] = zero16i
            npad = ((rcur + BATCH - 1) // BATCH) * BATCH

            def flush(j, carry):
                pltpu.sync_copy(res_s.at[pl.ds(j * FCH, FCH)],
                                rs_hbm.at[rid, s, pl.ds(j * FCH, FCH)])
                pltpu.sync_copy(res_d.at[pl.ds(j * FCH, FCH)],
                                rd_hbm.at[rid, s, pl.ds(j * FCH, FCH)])
                return carry

            lax.fori_loop(0, (npad + FCH - 1) // FCH, flush, jnp.int32(0))
            sm = iota16 == (s % 16)
            cv0 = jnp.where(sm & (s < 16), npad, cv0)
            cv1 = jnp.where(sm & (s >= 16), npad, cv1)
            return (cv0, cv1)

        cv0, cv1 = lax.fori_loop(0, NSP, span_step, (zero16i, zero16i))
        cv32[pl.ds(0, 16)] = cv0
        cv32[pl.ds(16, 16)] = cv1
        pltpu.sync_copy(cv32, cnts_hbm.at[rid])

        def mx(i, m):
            return jnp.maximum(m, cnt_l[pl.ds(i * 16, 16)])

        mv = lax.fori_loop(0, RNG // 16, mx, zero16i)
        kmax = jnp.maximum(jnp.max(mv), 1)
        kv[...] = jnp.where(lane0, kmax, 0)
        pltpu.sync_copy(kv, kmax_hbm.at[rid])
        pltpu.sync_copy(ell_st, ell_hbm.at[rid])
        pltpu.sync_copy(deg_l, degp_hbm.at[rid])


@functools.partial(
    pl.kernel,
    out_type=(
        jax.ShapeDtypeStruct((NRANGES, KMAX, 2, HGL), jnp.int32),
        jax.ShapeDtypeStruct((NRANGES, NSP, RESCAP), jnp.int32),
        jax.ShapeDtypeStruct((NRANGES, NSP, RESCAP), jnp.int32),
        jax.ShapeDtypeStruct((NRANGES, 32), jnp.int32),
        jax.ShapeDtypeStruct((NRANGES, 16), jnp.int32),
        jax.ShapeDtypeStruct((NRANGES, RNG), jnp.float32),
    ),
    mesh=_vector_mesh,
    scratch_types=[
        pltpu.VMEM((ES,), jnp.int32),
        pltpu.VMEM((ES,), jnp.int32),
        pltpu.VMEM((SELCAP,), jnp.int32),
        pltpu.VMEM((SELCAP,), jnp.int32),
        pltpu.VMEM((RESCAP,), jnp.int32),
        pltpu.VMEM((RESCAP,), jnp.int32),
        pltpu.VMEM((KMAX, 2, HGL), jnp.int32),
        pltpu.VMEM((RNG + 16,), jnp.int32),
        pltpu.VMEM((RNG,), jnp.float32),
        pltpu.VMEM((32,), jnp.int32),
        pltpu.VMEM((16,), jnp.int32),
    ],
    compiler_params=_sc_params,
)
def _bucket(*args):
    _bucket_body(*args)


NG = H // 128


def _agg_body(h_hbm, h4_hbm, ell_hbm, rs_hbm, rd_hbm, cnts_hbm, kmax_hbm,
              agg_hbm,
              acc, rows, ebuf, sebuf, sidx, didx, cv32, kv, semA):
    cid = lax.axis_index("c")
    sid = lax.axis_index("s")
    w = cid * NS + sid
    iota16 = lax.iota(jnp.int32, 16)
    zero16f = jnp.zeros((16,), jnp.float32)

    for r in range(NR):
        rid = r * NW + w
        lo = rid * RNG

        pltpu.sync_copy(kmax_hbm.at[rid], kv)
        kmax = jnp.max(jnp.where(iota16 == 0, kv[...], 0))

        @pl.loop(0, NG * RNG)
        def _(i):
            @pl.loop(0, 128 // 16)
            def _(q):
                acc[i, pl.ds(q * 16, 16)] = zero16f

        def chunk_step(c, carry, rid=rid):
            pltpu.sync_copy(ell_hbm.at[rid, pl.ds(c * CH, CH)], ebuf)
            nk = jnp.minimum(CH, kmax - c * CH)

            @pl.loop(0, CH)
            def _(k):
                @pl.loop(0, HGL // 16)
                def _(q):
                    for p in range(2):
                        e = ebuf[k, p, pl.ds(q * 16, 16)] * NG
                        for g in range(NG):
                            sebuf[pl.ds(k * (2 * NG * HGL)
                                        + (p * NG + g) * HGL
                                        + q * 16, 16)] = e + g

            def fire(k, c2):
                for p in range(2):
                    for g in range(NG):
                        pltpu.async_copy(
                            h4_hbm.at[sebuf.at[pl.ds(
                                k * (2 * NG * HGL) + (p * NG + g) * HGL,
                                HGL)]],
                            acc.at[pl.ds(g * RNG + p * HGL, HGL)],
                            semA, add=True)
                return c2

            lax.fori_loop(0, nk, fire, jnp.int32(0))

            def drain(k, c2):
                for p in range(2):
                    for g in range(NG):
                        pltpu.make_async_copy(
                            h4_hbm.at[pl.ds(0, HGL)],
                            acc.at[pl.ds(g * RNG + p * HGL, HGL)],
                            semA).wait()
                return c2

            lax.fori_loop(0, nk, drain, jnp.int32(0))
            return carry

        lax.fori_loop(0, (kmax + CH - 1) // CH, chunk_step, jnp.int32(0))

        pltpu.sync_copy(cnts_hbm.at[rid], cv32)

        def span_step(s, carry, rid=rid):
            nsel = cv32[pl.ds((s // 16) * 16, 16)]
            n = jnp.max(jnp.where(iota16 == s % 16, nsel, 0))

            def batch_step(j, c2):
                off = j * BATCH
                pltpu.sync_copy(rs_hbm.at[rid, s, pl.ds(off, BATCH)], sidx)
                pltpu.sync_copy(rd_hbm.at[rid, s, pl.ds(off, BATCH)],
                                didx.at[pl.ds(0, BATCH)])
                pltpu.async_copy(h_hbm.at[sidx], rows, semA).wait()

                def edge_step(e, c3):
                    ld = didx[pl.ds(e, 16)][0]
                    ldspl = jnp.full((16,), ld, jnp.int32)
                    for k in range(H // 16):
                        x = rows[e, pl.ds(k * 16, 16)]
                        g, q = k // 8, k % 8
                        col = iota16 + (q * 16)
                        plsc.addupdate_scatter(acc, [ldspl + g * RNG, col], x)
                    return c3

                lax.fori_loop(0, BATCH, edge_step, jnp.int32(0))
                return c2

            lax.fori_loop(0, n // BATCH, batch_step, jnp.int32(0))
            return carry

        lax.fori_loop(0, NSP, span_step, jnp.int32(0))
        for g in range(NG):
            pltpu.sync_copy(acc.at[pl.ds(g * RNG, RNG)],
                            agg_hbm.at[g, pl.ds(lo, RNG)])


@functools.partial(
    pl.kernel,
    out_type=jax.ShapeDtypeStruct((NG, AGGR, 128), jnp.float32),
    mesh=_vector_mesh,
    scratch_types=[
        pltpu.VMEM((NG * RNG, 128), jnp.float32),
        pltpu.VMEM((BATCH, H), jnp.float32),
        pltpu.VMEM((CH, 2, HGL), jnp.int32),
        pltpu.VMEM((CH * 2 * NG * HGL,), jnp.int32),
        pltpu.VMEM((BATCH,), jnp.int32),
        pltpu.VMEM((BATCH + 16,), jnp.int32),
        pltpu.VMEM((32,), jnp.int32),
        pltpu.VMEM((16,), jnp.int32),
        pltpu.SemaphoreType.DMA,
    ],
    compiler_params=_sc_params,
)
def _agg(*args):
    _agg_body(*args)


def _mm_stats_body(h_ref, a0_ref, a1_ref, a2_ref, a3_ref, dg_ref,
                   wl_ref, wr_ref, bc_ref, u_ref, stats_ref):
    i = pl.program_id(0)
    h = h_ref[...]
    dg = dg_ref[0, 0, :]
    a = jnp.concatenate([a0_ref[0], a1_ref[0], a2_ref[0], a3_ref[0]],
                        axis=1) * dg[:, None]
    u = lax.dot_general(h, wl_ref[...], (((1,), (1,)), ((), ())),
                        preferred_element_type=jnp.float32)
    u += lax.dot_general(a, wr_ref[...], (((1,), (1,)), ((), ())),
                         preferred_element_type=jnp.float32)
    u += bc_ref[...]
    u_ref[...] = u
    s1 = jnp.sum(u, axis=0, keepdims=True)
    s2 = jnp.sum(u * u, axis=0, keepdims=True)
    new = jnp.concatenate([s1, s2, jnp.zeros((6, H), jnp.float32)], axis=0)

    @pl.when(i == 0)
    def _():
        stats_ref[...] = new

    @pl.when(i > 0)
    def _():
        stats_ref[...] += new


def _mm_stats(h, agg4, deginv3d, wl, wr, bc):
    return pl.pallas_call(
        _mm_stats_body,
        grid=(NB,),
        in_specs=[
            pl.BlockSpec((R, H), lambda i: (i, 0)),
            pl.BlockSpec((1, R, 128), lambda i: (0, i, 0)),
            pl.BlockSpec((1, R, 128), lambda i: (1, i, 0)),
            pl.BlockSpec((1, R, 128), lambda i: (2, i, 0)),
            pl.BlockSpec((1, R, 128), lambda i: (3, i, 0)),
            pl.BlockSpec((1, 1, R), lambda i: (i, 0, 0)),
            pl.BlockSpec((H, H), lambda i: (0, 0)),
            pl.BlockSpec((H, H), lambda i: (0, 0)),
            pl.BlockSpec((1, H), lambda i: (0, 0)),
        ],
        out_specs=[
            pl.BlockSpec((R, H), lambda i: (i, 0)),
            pl.BlockSpec((8, H), lambda i: (0, 0)),
        ],
        out_shape=[
            jax.ShapeDtypeStruct((N, H), jnp.float32),
            jax.ShapeDtypeStruct((8, H), jnp.float32),
        ],
    )(h, agg4, agg4, agg4, agg4, deginv3d, wl, wr, bc)


def _bn_body(u_ref, stats_ref, g_ref, b_ref, o_ref):
    i = pl.program_id(0)

    @pl.when(i < NB)
    def _():
        u = u_ref[...]
        mu = stats_ref[0:1, :] * (1.0 / N)
        var = stats_ref[1:2, :] * (1.0 / N) - mu * mu
        inv = lax.rsqrt(var + 1e-5)
        o_ref[...] = jnp.maximum((u - mu) * inv * g_ref[...] + b_ref[...],
                                 0.0)

    @pl.when(i == NB)
    def _():
        o_ref[...] = jnp.zeros((R, H), jnp.float32)


def _bn_relu(u, stats, g, b):
    return pl.pallas_call(
        _bn_body,
        grid=(NB + 1,),
        in_specs=[
            pl.BlockSpec((R, H), lambda i: (jnp.minimum(i, NB - 1), 0)),
            pl.BlockSpec((8, H), lambda i: (0, 0)),
            pl.BlockSpec((1, H), lambda i: (0, 0)),
            pl.BlockSpec((1, H), lambda i: (0, 0)),
        ],
        out_specs=pl.BlockSpec((R, H), lambda i: (i, 0)),
        out_shape=jax.ShapeDtypeStruct((HPAD, H), jnp.float32),
    )(u, stats, g, b)


def _head_body(h_ref, batch_ref, fcw_ref, fcb_ref, w1_ref, b1_ref,
               w2_ref, b2_ref, o_ref, pooled_acc, cnt_acc):
    i = pl.program_id(0)
    b = batch_ref[0, 0, :]
    onehot = (b[:, None] == lax.broadcasted_iota(jnp.int32, (1, G), 1)
              ).astype(jnp.float32)
    pooled = lax.dot_general(onehot, h_ref[...], (((0,), (0,)), ((), ())),
                             preferred_element_type=jnp.float32)
    cnt = jnp.sum(onehot, axis=0, keepdims=True)

    @pl.when(i == 0)
    def _():
        pooled_acc[...] = pooled
        cnt_acc[...] = cnt

    @pl.when(i > 0)
    def _():
        pooled_acc[...] += pooled
        cnt_acc[...] += cnt

    @pl.when(i == NB - 1)
    def _():
        p = pooled_acc[...] / jnp.maximum(cnt_acc[...], 1.0).reshape(G, 1)
        t = lax.dot_general(p, fcw_ref[...], (((1,), (1,)), ((), ())),
                            preferred_element_type=jnp.float32)
        t = jnp.maximum(t + fcb_ref[...], 0.0)
        t = lax.dot_general(t, w1_ref[...], (((1,), (1,)), ((), ())),
                            preferred_element_type=jnp.float32)
        t = jnp.maximum(t + b1_ref[...], 0.0)
        t = lax.dot_general(t, w2_ref[...], (((1,), (1,)), ((), ())),
                            preferred_element_type=jnp.float32)
        o_ref[...] = t + b2_ref[...]


def _pool_head(h, batch3d, fc_W, fc_b, w1, b1, w2, b2):
    return pl.pallas_call(
        _head_body,
        grid=(NB,),
        in_specs=[
            pl.BlockSpec((R, H), lambda i: (i, 0)),
            pl.BlockSpec((1, 1, R), lambda i: (i, 0, 0)),
            pl.BlockSpec((H, H), lambda i: (0, 0)),
            pl.BlockSpec((1, H), lambda i: (0, 0)),
            pl.BlockSpec((H, H), lambda i: (0, 0)),
            pl.BlockSpec((1, H), lambda i: (0, 0)),
            pl.BlockSpec((OUT, H), lambda i: (0, 0)),
            pl.BlockSpec((1, OUT), lambda i: (0, 0)),
        ],
        out_specs=pl.BlockSpec((G, OUT), lambda i: (0, 0)),
        out_shape=jax.ShapeDtypeStruct((G, OUT), jnp.float32),
        scratch_shapes=[
            pltpu.VMEM((G, H), jnp.float32),
            pltpu.VMEM((1, G), jnp.float32),
        ],
    )(h, batch3d, fc_W, fc_b, w1, b1, w2, b2)


def kernel(x, gam0, gam1, gam2, edge_index, batch,
           Wl0, Wr0, bc0, bng0, bnb0,
           Wl1, Wr1, bc1, bng1, bnb1,
           Wl2, Wr2, bc2, bng2, bnb2,
           Wl3, Wr3, bc3, bng3, bnb3,
           fc_W, fc_b, mlp_W1, mlp_b1, mlp_W2, mlp_b2):
    h = jnp.concatenate([x, gam0, gam1, gam2], axis=1)
    h = jnp.concatenate([h, jnp.zeros((HPAD - N, H), jnp.float32)], axis=0)
    src = edge_index[0].astype(jnp.int32)
    dst = edge_index[1].astype(jnp.int32)
    batch3d = batch.astype(jnp.int32).reshape(NB, 1, R)

    ell, rs, rd, cnts, kmaxs, degp = _bucket(src, dst)
    deg = degp.reshape(AGGR)[:N]
    deginv3d = (1.0 / jnp.clip(deg, 1.0)).reshape(NB, 1, R)

    convs = [(Wl0, Wr0, bc0, bng0, bnb0), (Wl1, Wr1, bc1, bng1, bnb1),
             (Wl2, Wr2, bc2, bng2, bnb2), (Wl3, Wr3, bc3, bng3, bnb3)]
    for (Wl, Wr, bc, g, b) in convs:
        agg4 = _agg(h, h.reshape(HPAD * NG, 128), ell, rs, rd, cnts, kmaxs)
        u, stats = _mm_stats(h, agg4, deginv3d, Wl, Wr, bc.reshape(1, H))
        h = _bn_relu(u, stats, g.reshape(1, H), b.reshape(1, H))

    return _pool_head(h, batch3d, fc_W, fc_b.reshape(1, H),
                      mlp_W1, mlp_b1.reshape(1, H),
                      mlp_W2, mlp_b2.reshape(1, OUT))


# speedup vs baseline: 2.3266x; 2.2624x over previous
import dataclasses
import functools

import jax
import jax.numpy as jnp
from jax import lax
from jax.experimental import pallas as pl
from jax.experimental.pallas import tpu as pltpu
from jax.experimental.pallas import tpu_sc as plsc

N = 10000
E = 320000
H = 512
G = 16
OUT = 128
NB = 10
R = N // NB

NC = 2
NS = 16
NW = NC * NS
NSP = 32
ES = E // NSP
RNG = 160
NR = 2
NRANGES = NW * NR
AGGR = NRANGES * RNG
HGL = 80
KMAX = 1
CH = 1
HPAD = 11000
ZROW = N
BATCH = 64
SELCAP = ES + 256
RESCAP = 11264
FCH = 1024

_vector_mesh = plsc.VectorSubcoreMesh(core_axis_name="c", subcore_axis_name="s")

_sc_params = pltpu.CompilerParams()
if "needs_layout_passes" in pltpu.CompilerParams.__dataclass_fields__:
    _sc_params = dataclasses.replace(_sc_params, needs_layout_passes=False)


def _bucket_body(src_hbm, dst_hbm, ell_hbm, rs_hbm, rd_hbm, cnts_hbm,
                 kmax_hbm, degp_hbm,
                 sbuf, dbuf, sel_s, sel_d, res_s, res_d, ell_st, cnt_l,
                 deg_l, cv32, kv):
    cid = lax.axis_index("c")
    sid = lax.axis_index("s")
    w = cid * NS + sid

    zero16f = jnp.zeros((16,), jnp.float32)
    ones16 = jnp.ones((16,), jnp.float32)
    zero16i = jnp.zeros((16,), jnp.int32)
    zrow16 = jnp.full((16,), ZROW, jnp.int32)
    sent16 = jnp.full((16,), RNG, jnp.int32)
    iota16 = lax.iota(jnp.int32, 16)
    lane0 = iota16 == 0

    for r in range(NR):
        rid = r * NW + w
        lo = rid * RNG

        @pl.loop(0, RNG // 16)
        def _(i):
            deg_l[pl.ds(i * 16, 16)] = zero16f

        @pl.loop(0, (RNG + 16) // 16)
        def _(i):
            cnt_l[pl.ds(i * 16, 16)] = zero16i

        @pl.loop(0, KMAX)
        def _(k):
            @pl.loop(0, 2)
            def _(p):
                @pl.loop(0, HGL // 16)
                def _(q):
                    ell_st[k, p, pl.ds(q * 16, 16)] = zrow16

        def span_step(s, cvs, lo=lo, rid=rid):
            cv0, cv1 = cvs
            pltpu.sync_copy(src_hbm.at[pl.ds(s * ES, ES)], sbuf)
            pltpu.sync_copy(dst_hbm.at[pl.ds(s * ES, ES)], dbuf)

            def step(i, cur):
                sv = sbuf[pl.ds(i * 16, 16)]
                dv = dbuf[pl.ds(i * 16, 16)] - lo
                m = (dv >= 0) & (dv < RNG)
                plsc.store_compressed(sel_s.at[pl.ds(cur, 16)], sv, mask=m)
                plsc.store_compressed(sel_d.at[pl.ds(cur, 16)], dv, mask=m)
                plsc.addupdate_scatter(deg_l, [dv], ones16, mask=m)
                return cur + jnp.max(plsc.all_reduce_population_count(m))

            cur = lax.fori_loop(0, ES // 16, step, jnp.int32(0))

            def walk(e, rcur):
                ld = sel_d[pl.ds(e, 16)][0]
                sv = sel_s[pl.ds(e, 16)][0]
                c = cnt_l[pl.ds(ld, 16)][0]
                ldspl = jnp.full((16,), ld, jnp.int32)
                svspl = jnp.full((16,), sv, jnp.int32)

                @pl.when(c < KMAX)
                def _():
                    ph = ld // HGL
                    col = ld - ph * HGL
                    plsc.store_scatter(
                        ell_st,
                        [jnp.full((16,), c, jnp.int32),
                         jnp.full((16,), ph, jnp.int32),
                         jnp.full((16,), col, jnp.int32)],
                        svspl, mask=lane0)
                    plsc.store_scatter(
                        cnt_l, [ldspl], jnp.full((16,), c + 1, jnp.int32),
                        mask=lane0)

                @pl.when(c >= KMAX)
                def _():
                    plsc.store_compressed(res_s.at[pl.ds(rcur, 16)], svspl,
                                          mask=lane0)
                    plsc.store_compressed(res_d.at[pl.ds(rcur, 16)], ldspl,
                                          mask=lane0)

                return rcur + jnp.where(c >= KMAX, 1, 0).astype(jnp.int32)

            rcur = lax.fori_loop(0, cur, walk, jnp.int32(0))

            for k in range(BATCH // 16):
                res_s[pl.ds(rcur + k * 16, 16)] = zrow16
                res_d[pl.ds(rcur + k * 16, 16)] = zero16i
            npad = ((rcur + BATCH - 1) // BATCH) * BATCH

            def flush(j, carry):
                pltpu.sync_copy(res_s.at[pl.ds(j * FCH, FCH)],
                                rs_hbm.at[rid, s, pl.ds(j * FCH, FCH)])
                pltpu.sync_copy(res_d.at[pl.ds(j * FCH, FCH)],
                                rd_hbm.at[rid, s, pl.ds(j * FCH, FCH)])
                return carry

            lax.fori_loop(0, (npad + FCH - 1) // FCH, flush, jnp.int32(0))
            sm = iota16 == (s % 16)
            cv0 = jnp.where(sm & (s < 16), npad, cv0)
            cv1 = jnp.where(sm & (s >= 16), npad, cv1)
            return (cv0, cv1)

        cv0, cv1 = lax.fori_loop(0, NSP, span_step, (zero16i, zero16i))
        cv32[pl.ds(0, 16)] = cv0
        cv32[pl.ds(16, 16)] = cv1
        pltpu.sync_copy(cv32, cnts_hbm.at[rid])

        def mx(i, m):
            return jnp.maximum(m, cnt_l[pl.ds(i * 16, 16)])

        mv = lax.fori_loop(0, RNG // 16, mx, zero16i)
        kmax = jnp.maximum(jnp.max(mv), 1)
        kv[...] = jnp.where(lane0, kmax, 0)
        pltpu.sync_copy(kv, kmax_hbm.at[rid])
        pltpu.sync_copy(ell_st, ell_hbm.at[rid])
        pltpu.sync_copy(deg_l, degp_hbm.at[rid])


@functools.partial(
    pl.kernel,
    out_type=(
        jax.ShapeDtypeStruct((NRANGES, KMAX, 2, HGL), jnp.int32),
        jax.ShapeDtypeStruct((NRANGES, NSP, RESCAP), jnp.int32),
        jax.ShapeDtypeStruct((NRANGES, NSP, RESCAP), jnp.int32),
        jax.ShapeDtypeStruct((NRANGES, 32), jnp.int32),
        jax.ShapeDtypeStruct((NRANGES, 16), jnp.int32),
        jax.ShapeDtypeStruct((NRANGES, RNG), jnp.float32),
    ),
    mesh=_vector_mesh,
    scratch_types=[
        pltpu.VMEM((ES,), jnp.int32),
        pltpu.VMEM((ES,), jnp.int32),
        pltpu.VMEM((SELCAP,), jnp.int32),
        pltpu.VMEM((SELCAP,), jnp.int32),
        pltpu.VMEM((RESCAP,), jnp.int32),
        pltpu.VMEM((RESCAP,), jnp.int32),
        pltpu.VMEM((KMAX, 2, HGL), jnp.int32),
        pltpu.VMEM((RNG + 16,), jnp.int32),
        pltpu.VMEM((RNG,), jnp.float32),
        pltpu.VMEM((32,), jnp.int32),
        pltpu.VMEM((16,), jnp.int32),
    ],
    compiler_params=_sc_params,
)
def _bucket(*args):
    _bucket_body(*args)


NG = H // 128


def _agg_body(h_hbm, h4_hbm, ell_hbm, rs_hbm, rd_hbm, cnts_hbm, kmax_hbm,
              agg_hbm,
              acc, rows, ebuf, sebuf, sidx, didx, cv32, kv, semA):
    cid = lax.axis_index("c")
    sid = lax.axis_index("s")
    w = cid * NS + sid
    iota16 = lax.iota(jnp.int32, 16)
    zero16f = jnp.zeros((16,), jnp.float32)

    for r in range(NR):
        rid = r * NW + w
        lo = rid * RNG

        pltpu.sync_copy(kmax_hbm.at[rid], kv)
        kmax = jnp.max(jnp.where(iota16 == 0, kv[...], 0))

        @pl.loop(0, NG * RNG)
        def _(i):
            @pl.loop(0, 128 // 16)
            def _(q):
                acc[i, pl.ds(q * 16, 16)] = zero16f

        def chunk_step(c, carry, rid=rid):
            pltpu.sync_copy(ell_hbm.at[rid, pl.ds(c * CH, CH)], ebuf)
            nk = jnp.minimum(CH, kmax - c * CH)

            @pl.loop(0, CH)
            def _(k):
                @pl.loop(0, HGL // 16)
                def _(q):
                    for p in range(2):
                        e = ebuf[k, p, pl.ds(q * 16, 16)] * NG
                        for g in range(NG):
                            sebuf[pl.ds(k * (2 * NG * HGL)
                                        + (p * NG + g) * HGL
                                        + q * 16, 16)] = e + g

            def fire(k, c2):
                for p in range(2):
                    for g in range(NG):
                        pltpu.async_copy(
                            h4_hbm.at[sebuf.at[pl.ds(
                                k * (2 * NG * HGL) + (p * NG + g) * HGL,
                                HGL)]],
                            acc.at[pl.ds(g * RNG + p * HGL, HGL)],
                            semA, add=True)
                return c2

            lax.fori_loop(0, nk, fire, jnp.int32(0))

            def drain(k, c2):
                for p in range(2):
                    for g in range(NG):
                        pltpu.make_async_copy(
                            h4_hbm.at[pl.ds(0, HGL)],
                            acc.at[pl.ds(g * RNG + p * HGL, HGL)],
                            semA).wait()
                return c2

            lax.fori_loop(0, nk, drain, jnp.int32(0))
            return carry

        lax.fori_loop(0, (kmax + CH - 1) // CH, chunk_step, jnp.int32(0))

        pltpu.sync_copy(cnts_hbm.at[rid], cv32)

        def span_step(s, carry, rid=rid):
            nsel = cv32[pl.ds((s // 16) * 16, 16)]
            n = jnp.max(jnp.where(iota16 == s % 16, nsel, 0))

            def batch_step(j, c2):
                off = j * BATCH
                pltpu.sync_copy(rs_hbm.at[rid, s, pl.ds(off, BATCH)], sidx)
                pltpu.sync_copy(rd_hbm.at[rid, s, pl.ds(off, BATCH)],
                                didx.at[pl.ds(0, BATCH)])
                pltpu.async_copy(h_hbm.at[sidx], rows, semA).wait()

                def edge_step(e, c3):
                    ld = didx[pl.ds(e, 16)][0]
                    ldspl = jnp.full((16,), ld, jnp.int32)
                    for k in range(H // 16):
                        x = rows[e, pl.ds(k * 16, 16)]
                        g, q = k // 8, k % 8
                        col = iota16 + (q * 16)
                        plsc.addupdate_scatter(acc, [ldspl + g * RNG, col], x)
                    return c3

                lax.fori_loop(0, BATCH, edge_step, jnp.int32(0))
                return c2

            lax.fori_loop(0, n // BATCH, batch_step, jnp.int32(0))
            return carry

        lax.fori_loop(0, NSP, span_step, jnp.int32(0))
        for g in range(NG):
            pltpu.sync_copy(acc.at[pl.ds(g * RNG, RNG)],
                            agg_hbm.at[g, pl.ds(lo, RNG)])


@functools.partial(
    pl.kernel,
    out_type=jax.ShapeDtypeStruct((NG, AGGR, 128), jnp.float32),
    mesh=_vector_mesh,
    scratch_types=[
        pltpu.VMEM((NG * RNG, 128), jnp.float32),
        pltpu.VMEM((BATCH, H), jnp.float32),
        pltpu.VMEM((CH, 2, HGL), jnp.int32),
        pltpu.VMEM((CH * 2 * NG * HGL,), jnp.int32),
        pltpu.VMEM((BATCH,), jnp.int32),
        pltpu.VMEM((BATCH + 16,), jnp.int32),
        pltpu.VMEM((32,), jnp.int32),
        pltpu.VMEM((16,), jnp.int32),
        pltpu.SemaphoreType.DMA,
    ],
    compiler_params=_sc_params,
)
def _agg(*args):
    _agg_body(*args)


def _mm_stats_body(h_ref, a0_ref, a1_ref, a2_ref, a3_ref, dg_ref,
                   wl_ref, wr_ref, bc_ref, u_ref, stats_ref):
    i = pl.program_id(0)
    h = h_ref[...]
    dg = dg_ref[0, 0, :]
    a = jnp.concatenate([a0_ref[0], a1_ref[0], a2_ref[0], a3_ref[0]],
                        axis=1) * dg[:, None]
    u = lax.dot_general(h, wl_ref[...], (((1,), (1,)), ((), ())),
                        preferred_element_type=jnp.float32)
    u += lax.dot_general(a, wr_ref[...], (((1,), (1,)), ((), ())),
                         preferred_element_type=jnp.float32)
    u += bc_ref[...]
    u_ref[...] = u
    s1 = jnp.sum(u, axis=0, keepdims=True)
    s2 = jnp.sum(u * u, axis=0, keepdims=True)
    new = jnp.concatenate([s1, s2, jnp.zeros((6, H), jnp.float32)], axis=0)

    @pl.when(i == 0)
    def _():
        stats_ref[...] = new

    @pl.when(i > 0)
    def _():
        stats_ref[...] += new


def _mm_stats(h, agg4, deginv3d, wl, wr, bc):
    return pl.pallas_call(
        _mm_stats_body,
        grid=(NB,),
        in_specs=[
            pl.BlockSpec((R, H), lambda i: (i, 0)),
            pl.BlockSpec((1, R, 128), lambda i: (0, i, 0)),
            pl.BlockSpec((1, R, 128), lambda i: (1, i, 0)),
            pl.BlockSpec((1, R, 128), lambda i: (2, i, 0)),
            pl.BlockSpec((1, R, 128), lambda i: (3, i, 0)),
            pl.BlockSpec((1, 1, R), lambda i: (i, 0, 0)),
            pl.BlockSpec((H, H), lambda i: (0, 0)),
            pl.BlockSpec((H, H), lambda i: (0, 0)),
            pl.BlockSpec((1, H), lambda i: (0, 0)),
        ],
        out_specs=[
            pl.BlockSpec((R, H), lambda i: (i, 0)),
            pl.BlockSpec((8, H), lambda i: (0, 0)),
        ],
        out_shape=[
            jax.ShapeDtypeStruct((N, H), jnp.float32),
            jax.ShapeDtypeStruct((8, H), jnp.float32),
        ],
    )(h, agg4, agg4, agg4, agg4, deginv3d, wl, wr, bc)


def _bn_body(u_ref, stats_ref, g_ref, b_ref, o_ref):
    i = pl.program_id(0)

    @pl.when(i < NB)
    def _():
        u = u_ref[...]
        mu = stats_ref[0:1, :] * (1.0 / N)
        var = stats_ref[1:2, :] * (1.0 / N) - mu * mu
        inv = lax.rsqrt(var + 1e-5)
        o_ref[...] = jnp.maximum((u - mu) * inv * g_ref[...] + b_ref[...],
                                 0.0)

    @pl.when(i == NB)
    def _():
        o_ref[...] = jnp.zeros((R, H), jnp.float32)


def _bn_relu(u, stats, g, b):
    return pl.pallas_call(
        _bn_body,
        grid=(NB + 1,),
        in_specs=[
            pl.BlockSpec((R, H), lambda i: (jnp.minimum(i, NB - 1), 0)),
            pl.BlockSpec((8, H), lambda i: (0, 0)),
            pl.BlockSpec((1, H), lambda i: (0, 0)),
            pl.BlockSpec((1, H), lambda i: (0, 0)),
        ],
        out_specs=pl.BlockSpec((R, H), lambda i: (i, 0)),
        out_shape=jax.ShapeDtypeStruct((HPAD, H), jnp.float32),
    )(u, stats, g, b)


def _head_body(h_ref, batch_ref, fcw_ref, fcb_ref, w1_ref, b1_ref,
               w2_ref, b2_ref, o_ref, pooled_acc, cnt_acc):
    i = pl.program_id(0)
    b = batch_ref[0, 0, :]
    onehot = (b[:, None] == lax.broadcasted_iota(jnp.int32, (1, G), 1)
              ).astype(jnp.float32)
    pooled = lax.dot_general(onehot, h_ref[...], (((0,), (0,)), ((), ())),
                             preferred_element_type=jnp.float32)
    cnt = jnp.sum(onehot, axis=0, keepdims=True)

    @pl.when(i == 0)
    def _():
        pooled_acc[...] = pooled
        cnt_acc[...] = cnt

    @pl.when(i > 0)
    def _():
        pooled_acc[...] += pooled
        cnt_acc[...] += cnt

    @pl.when(i == NB - 1)
    def _():
        p = pooled_acc[...] / jnp.maximum(cnt_acc[...], 1.0).reshape(G, 1)
        t = lax.dot_general(p, fcw_ref[...], (((1,), (1,)), ((), ())),
                            preferred_element_type=jnp.float32)
        t = jnp.maximum(t + fcb_ref[...], 0.0)
        t = lax.dot_general(t, w1_ref[...], (((1,), (1,)), ((), ())),
                            preferred_element_type=jnp.float32)
        t = jnp.maximum(t + b1_ref[...], 0.0)
        t = lax.dot_general(t, w2_ref[...], (((1,), (1,)), ((), ())),
                            preferred_element_type=jnp.float32)
        o_ref[...] = t + b2_ref[...]


def _pool_head(h, batch3d, fc_W, fc_b, w1, b1, w2, b2):
    return pl.pallas_call(
        _head_body,
        grid=(NB,),
        in_specs=[
            pl.BlockSpec((R, H), lambda i: (i, 0)),
            pl.BlockSpec((1, 1, R), lambda i: (i, 0, 0)),
            pl.BlockSpec((H, H), lambda i: (0, 0)),
            pl.BlockSpec((1, H), lambda i: (0, 0)),
            pl.BlockSpec((H, H), lambda i: (0, 0)),
            pl.BlockSpec((1, H), lambda i: (0, 0)),
            pl.BlockSpec((OUT, H), lambda i: (0, 0)),
            pl.BlockSpec((1, OUT), lambda i: (0, 0)),
        ],
        out_specs=pl.BlockSpec((G, OUT), lambda i: (0, 0)),
        out_shape=jax.ShapeDtypeStruct((G, OUT), jnp.float32),
        scratch_shapes=[
            pltpu.VMEM((G, H), jnp.float32),
            pltpu.VMEM((1, G), jnp.float32),
        ],
    )(h, batch3d, fc_W, fc_b, w1, b1, w2, b2)


def kernel(x, gam0, gam1, gam2, edge_index, batch,
           Wl0, Wr0, bc0, bng0, bnb0,
           Wl1, Wr1, bc1, bng1, bnb1,
           Wl2, Wr2, bc2, bng2, bnb2,
           Wl3, Wr3, bc3, bng3, bnb3,
           fc_W, fc_b, mlp_W1, mlp_b1, mlp_W2, mlp_b2):
    h = jnp.concatenate([x, gam0, gam1, gam2], axis=1)
    h = jnp.concatenate([h, jnp.zeros((HPAD - N, H), jnp.float32)], axis=0)
    src = edge_index[0].astype(jnp.int32)
    dst = edge_index[1].astype(jnp.int32)
    batch3d = batch.astype(jnp.int32).reshape(NB, 1, R)

    ell, rs, rd, cnts, kmaxs, degp = _bucket(src, dst)
    deg = degp.reshape(AGGR)[:N]
    deginv3d = (1.0 / jnp.clip(deg, 1.0)).reshape(NB, 1, R)

    convs = [(Wl0, Wr0, bc0, bng0, bnb0), (Wl1, Wr1, bc1, bng1, bnb1),
             (Wl2, Wr2, bc2, bng2, bnb2), (Wl3, Wr3, bc3, bng3, bnb3)]
    for (Wl, Wr, bc, g, b) in convs:
        agg4 = _agg(h, h.reshape(HPAD * NG, 128), ell, rs, rd, cnts, kmaxs)
        u, stats = _mm_stats(h, agg4, deginv3d, Wl, Wr, bc.reshape(1, H))
        h = _bn_relu(u, stats, g.reshape(1, H), b.reshape(1, H))

    return _pool_head(h, batch3d, fc_W, fc_b.reshape(1, H),
                      mlp_W1, mlp_b1.reshape(1, H),
                      mlp_W2, mlp_b2.reshape(1, OUT))
